# baseline (speedup 1.0000x reference)
import functools

import jax
import jax.numpy as jnp
from jax import lax
from jax.experimental import pallas as pl
from jax.experimental.pallas import tpu as pltpu
from jax.experimental.pallas import tpu_sc as plsc

BAR_VOC = 512
POS_VOC = 128

_VEL_OFF = 0
_DUR_OFF = 32
_BAR_OFF = 96
_POS_OFF = 608
_SMALL_ROWS = 736


def _word_proj_body(w_ref, wp_ref, o_ref):
    w1 = wp_ref[...][0:64, :]
    x = w_ref[...]
    o_ref[...] = jnp.concatenate(
        [jnp.dot(x[:, 64 * i:64 * (i + 1)], w1,
                 preferred_element_type=jnp.float32) for i in range(4)],
        axis=1)


def _word_proj(word_emb, W_proj):
    n_token, d = word_emb.shape
    word2 = word_emb.reshape(n_token // 4, 4 * d)
    blk = 1000
    grid = (word2.shape[0] // blk,)
    out = pl.pallas_call(
        _word_proj_body,
        grid=grid,
        in_specs=[
            pl.BlockSpec((blk, 4 * d), lambda i: (i, 0)),
            pl.BlockSpec(W_proj.shape, lambda i: (0, 0)),
        ],
        out_specs=pl.BlockSpec((blk, 4 * d), lambda i: (i, 0)),
        out_shape=jax.ShapeDtypeStruct(word2.shape, jnp.float32),
    )(word2, W_proj)
    return out.reshape(n_token, d)


def _small_proj_body(vel_ref, dur_ref, bar_ref, pos_ref, wp_ref, b_ref, o_ref):
    W = wp_ref[...]
    dot = functools.partial(jnp.dot, preferred_element_type=jnp.float32)
    o_ref[_VEL_OFF:_DUR_OFF, :] = dot(vel_ref[...], W[64:128, :]) + b_ref[...]
    o_ref[_DUR_OFF:_BAR_OFF, :] = dot(dur_ref[...], W[128:192, :])
    o_ref[_BAR_OFF:_POS_OFF, :] = dot(bar_ref[...], W[192:256, :])
    o_ref[_POS_OFF:_SMALL_ROWS, :] = dot(pos_ref[...], W[256:320, :])


def _small_proj(vel_emb, dur_emb, bar_embed, pos_embed, W_proj, b_proj):
    d = vel_emb.shape[1]
    return pl.pallas_call(
        _small_proj_body,
        out_shape=jax.ShapeDtypeStruct((_SMALL_ROWS, d), jnp.float32),
    )(vel_emb, dur_emb, bar_embed, pos_embed, W_proj, b_proj.reshape(1, d))


def _sc_gather_sum(word_proj, small_proj, tok, sidx):
    n_token, d = word_proj.shape
    BL = tok.shape[0]
    info = plsc.get_sparse_core_info()
    nw = info.num_cores * info.num_subcores
    per_w = BL // nw
    C = 256
    nk = C // 128
    n_chunks = per_w // C
    assert per_w % C == 0 and BL % (128 * nw) == 0

    mesh = plsc.VectorSubcoreMesh(core_axis_name="c", subcore_axis_name="s")

    @functools.partial(
        pl.kernel,
        out_type=jax.ShapeDtypeStruct((BL, d), jnp.float32),
        mesh=mesh,
        compiler_params=pltpu.CompilerParams(needs_layout_passes=False,
                                             use_tc_tiling_on_sc=False),
        scratch_types=[
            pltpu.VMEM((_SMALL_ROWS, d), jnp.float32),
            pltpu.VMEM((2, nk, 128), jnp.int32),
            pltpu.VMEM((2, 4, C + 16), jnp.int32),
            pltpu.VMEM((2, C, d), jnp.float32),
            pltpu.SemaphoreType.DMA,
            pltpu.SemaphoreType.DMA,
            pltpu.SemaphoreType.DMA,
        ],
    )
    def body(wordp_hbm, smallp_hbm, tok_hbm, sidx_hbm,
             out_hbm, small_v, tok_v, sidx_v, rows_v, sem_i, sem_g, sem_o):
        wid = lax.axis_index("s") * info.num_cores + lax.axis_index("c")
        base = wid * per_w
        pltpu.sync_copy(smallp_hbm, small_v)

        def issue_idx(g):
            buf = g & 1
            cbase = base + g * C
            for j in range(nk):
                pltpu.async_copy(tok_hbm.at[pl.ds(cbase + j * 128, 128)],
                                 tok_v.at[buf, j], sem_i)
            pltpu.async_copy(sidx_hbm.at[:, pl.ds(cbase, C)],
                             sidx_v.at[buf, :, pl.ds(0, C)], sem_i)

        def wait_idx(g):
            buf = g & 1
            for j in range(nk):
                pltpu.make_async_copy(tok_hbm.at[pl.ds(0, 128)],
                                      tok_v.at[buf, j], sem_i).wait()
            pltpu.make_async_copy(sidx_hbm.at[:, pl.ds(0, C)],
                                  sidx_v.at[buf, :, pl.ds(0, C)], sem_i).wait()

        def issue_gather(g):
            buf = g & 1
            for j in range(nk):
                pltpu.async_copy(wordp_hbm.at[tok_v.at[buf, j]],
                                 rows_v.at[buf, pl.ds(j * 128, 128)], sem_g)

        def wait_gather(g):
            buf = g & 1
            for j in range(nk):
                pltpu.make_async_copy(wordp_hbm.at[pl.ds(0, 128)],
                                      rows_v.at[buf, pl.ds(j * 128, 128)],
                                      sem_g).wait()

        def issue_out(g):
            buf = g & 1
            pltpu.async_copy(rows_v.at[buf],
                             out_hbm.at[pl.ds(base + g * C, C)], sem_o)

        def wait_out(g):
            buf = g & 1
            pltpu.make_async_copy(rows_v.at[buf],
                                  out_hbm.at[pl.ds(0, C)], sem_o).wait()

        issue_idx(0)
        wait_idx(0)
        issue_gather(0)
        issue_idx(1)

        def chunk_body(g, carry):
            buf = g & 1

            @pl.when(g + 1 < n_chunks)
            def _():
                wait_idx(g + 1)

                @pl.when(g >= 1)
                def _():
                    wait_out(g - 1)

                issue_gather(g + 1)

            wait_gather(g)

            def pos_body(p, c2):
                rs = [sidx_v[buf, t, pl.ds(p, 16)][0] for t in range(4)]
                accs = [rows_v[buf, p, pl.ds(16 * k, 16)]
                        for k in range(d // 16)]
                for t in range(4):
                    for k in range(d // 16):
                        accs[k] = accs[k] + small_v[rs[t], pl.ds(16 * k, 16)]
                for k in range(d // 16):
                    rows_v[buf, p, pl.ds(16 * k, 16)] = accs[k]
                return c2

            lax.fori_loop(0, C, pos_body, 0, unroll=4)

            @pl.when(g + 2 < n_chunks)
            def _():
                issue_idx(g + 2)

            issue_out(g)
            return carry

        lax.fori_loop(0, n_chunks, chunk_body, 0)
        wait_out(n_chunks - 2)
        wait_out(n_chunks - 1)

    return body(word_proj, small_proj, tok, sidx)


def kernel(token, vel, dur, bar, pos, word_emb, vel_emb, dur_emb, bar_embed,
           pos_embed, W_proj, b_proj):
    B, L = token.shape
    d = word_emb.shape[1]
    BL = B * L

    word_proj = _word_proj(word_emb, W_proj)
    small_proj = _small_proj(vel_emb, dur_emb, bar_embed, pos_embed,
                             W_proj, b_proj)

    tok = token.reshape(BL).astype(jnp.int32)
    vi = vel.reshape(BL).astype(jnp.int32) + _VEL_OFF
    di = dur.reshape(BL).astype(jnp.int32) + _DUR_OFF
    bi = (bar.astype(jnp.int32) % BAR_VOC).reshape(BL) + _BAR_OFF
    pi = pos.reshape(BL).astype(jnp.int32) + _POS_OFF
    sidx = jnp.stack([vi, di, bi, pi], axis=0)

    out = _sc_gather_sum(word_proj, small_proj, tok, sidx)
    return out.reshape(B, L, d)


# speedup vs baseline: 1.0994x; 1.0994x over previous
import functools

import jax
import jax.numpy as jnp
from jax import lax
from jax.experimental import pallas as pl
from jax.experimental.pallas import tpu as pltpu
from jax.experimental.pallas import tpu_sc as plsc

BAR_VOC = 512
POS_VOC = 128

_VEL_OFF = 0
_DUR_OFF = 32
_BAR_OFF = 96
_POS_OFF = 608
_SMALL_ROWS = 736


def _word_proj_body(w_ref, wp_ref, o_ref):
    w1 = wp_ref[...][0:64, :]
    x = w_ref[...]
    o_ref[...] = jnp.concatenate(
        [jnp.dot(x[:, 64 * i:64 * (i + 1)], w1,
                 preferred_element_type=jnp.float32) for i in range(4)],
        axis=1)


def _word_proj(word_emb, W_proj):
    n_token, d = word_emb.shape
    word2 = word_emb.reshape(n_token // 4, 4 * d)
    blk = 1000
    grid = (word2.shape[0] // blk,)
    out = pl.pallas_call(
        _word_proj_body,
        grid=grid,
        in_specs=[
            pl.BlockSpec((blk, 4 * d), lambda i: (i, 0)),
            pl.BlockSpec(W_proj.shape, lambda i: (0, 0)),
        ],
        out_specs=pl.BlockSpec((blk, 4 * d), lambda i: (i, 0)),
        out_shape=jax.ShapeDtypeStruct(word2.shape, jnp.float32),
    )(word2, W_proj)
    return out.reshape(n_token, d)


def _small_proj_body(vel_ref, dur_ref, bar_ref, pos_ref, wp_ref, b_ref, o_ref):
    W = wp_ref[...]
    dot = functools.partial(jnp.dot, preferred_element_type=jnp.float32)
    o_ref[_VEL_OFF:_DUR_OFF, :] = dot(vel_ref[...], W[64:128, :]) + b_ref[...]
    o_ref[_DUR_OFF:_BAR_OFF, :] = dot(dur_ref[...], W[128:192, :])
    o_ref[_BAR_OFF:_POS_OFF, :] = dot(bar_ref[...], W[192:256, :])
    o_ref[_POS_OFF:_SMALL_ROWS, :] = dot(pos_ref[...], W[256:320, :])


def _small_proj(vel_emb, dur_emb, bar_embed, pos_embed, W_proj, b_proj):
    d = vel_emb.shape[1]
    return pl.pallas_call(
        _small_proj_body,
        out_shape=jax.ShapeDtypeStruct((_SMALL_ROWS, d), jnp.float32),
    )(vel_emb, dur_emb, bar_embed, pos_embed, W_proj, b_proj.reshape(1, d))


def _sc_gather_sum(word_proj, small_proj, tok, sidx):
    n_token, d = word_proj.shape
    BL = tok.shape[0]
    info = plsc.get_sparse_core_info()
    nw = info.num_cores * info.num_subcores
    per_w = BL // nw
    C = 256
    nk = C // 128
    n_chunks = per_w // C
    assert per_w % C == 0 and BL % (128 * nw) == 0

    mesh = plsc.VectorSubcoreMesh(core_axis_name="c", subcore_axis_name="s")

    @functools.partial(
        pl.kernel,
        out_type=jax.ShapeDtypeStruct((BL, d), jnp.float32),
        mesh=mesh,
        compiler_params=pltpu.CompilerParams(needs_layout_passes=False,
                                             use_tc_tiling_on_sc=False),
        scratch_types=[
            pltpu.VMEM((_SMALL_ROWS, d), jnp.float32),
            pltpu.VMEM((2, nk, 128), jnp.int32),
            pltpu.VMEM((2, 4, C + 16), jnp.int32),
            pltpu.VMEM((2, C, d), jnp.float32),
            pltpu.SemaphoreType.DMA,
            pltpu.SemaphoreType.DMA,
            pltpu.SemaphoreType.DMA,
        ],
    )
    def body(wordp_hbm, smallp_hbm, tok_hbm, sidx_hbm,
             out_hbm, small_v, tok_v, sidx_v, rows_v, sem_i, sem_g, sem_o):
        wid = lax.axis_index("s") * info.num_cores + lax.axis_index("c")
        base = wid * per_w
        pltpu.sync_copy(smallp_hbm, small_v)

        def issue_idx(g):
            buf = g & 1
            cbase = base + g * C
            for j in range(nk):
                pltpu.async_copy(tok_hbm.at[pl.ds(cbase + j * 128, 128)],
                                 tok_v.at[buf, j], sem_i)
            pltpu.async_copy(sidx_hbm.at[:, pl.ds(cbase, C)],
                             sidx_v.at[buf, :, pl.ds(0, C)], sem_i)

        def wait_idx(g):
            buf = g & 1
            for j in range(nk):
                pltpu.make_async_copy(tok_hbm.at[pl.ds(0, 128)],
                                      tok_v.at[buf, j], sem_i).wait()
            pltpu.make_async_copy(sidx_hbm.at[:, pl.ds(0, C)],
                                  sidx_v.at[buf, :, pl.ds(0, C)], sem_i).wait()

        def issue_gather(g):
            buf = g & 1
            for j in range(nk):
                pltpu.async_copy(wordp_hbm.at[tok_v.at[buf, j]],
                                 rows_v.at[buf, pl.ds(j * 128, 128)], sem_g)

        def wait_gather(g):
            buf = g & 1
            for j in range(nk):
                pltpu.make_async_copy(wordp_hbm.at[pl.ds(0, 128)],
                                      rows_v.at[buf, pl.ds(j * 128, 128)],
                                      sem_g).wait()

        def issue_out(g):
            buf = g & 1
            pltpu.async_copy(rows_v.at[buf],
                             out_hbm.at[pl.ds(base + g * C, C)], sem_o)

        def wait_out(g):
            buf = g & 1
            pltpu.make_async_copy(rows_v.at[buf],
                                  out_hbm.at[pl.ds(0, C)], sem_o).wait()

        issue_idx(0)
        wait_idx(0)
        issue_gather(0)
        issue_idx(1)

        def chunk_body(g, carry):
            buf = g & 1

            @pl.when(g + 1 < n_chunks)
            def _():
                wait_idx(g + 1)

                @pl.when(g >= 1)
                def _():
                    wait_out(g - 1)

                issue_gather(g + 1)

            wait_gather(g)
            bufs = jnp.full((16,), buf, jnp.int32)
            cols = [lax.iota(jnp.int32, 16) + 16 * k for k in range(d // 16)]

            def pos_body(p, c2):
                splat = jnp.full((16,), p, jnp.int32)
                accs = [rows_v[buf, p, pl.ds(16 * k, 16)]
                        for k in range(d // 16)]
                for t in range(4):
                    r = plsc.load_gather(
                        sidx_v, [bufs, jnp.full((16,), t, jnp.int32), splat])
                    for k in range(d // 16):
                        accs[k] = accs[k] + plsc.load_gather(small_v,
                                                             [r, cols[k]])
                for k in range(d // 16):
                    rows_v[buf, p, pl.ds(16 * k, 16)] = accs[k]
                return c2

            lax.fori_loop(0, C, pos_body, 0, unroll=8)

            @pl.when(g + 2 < n_chunks)
            def _():
                issue_idx(g + 2)

            issue_out(g)
            return carry

        lax.fori_loop(0, n_chunks, chunk_body, 0)
        wait_out(n_chunks - 2)
        wait_out(n_chunks - 1)

    return body(word_proj, small_proj, tok, sidx)


def kernel(token, vel, dur, bar, pos, word_emb, vel_emb, dur_emb, bar_embed,
           pos_embed, W_proj, b_proj):
    B, L = token.shape
    d = word_emb.shape[1]
    BL = B * L

    word_proj = _word_proj(word_emb, W_proj)
    small_proj = _small_proj(vel_emb, dur_emb, bar_embed, pos_embed,
                             W_proj, b_proj)

    tok = token.reshape(BL).astype(jnp.int32)
    vi = vel.reshape(BL).astype(jnp.int32) + _VEL_OFF
    di = dur.reshape(BL).astype(jnp.int32) + _DUR_OFF
    bi = (bar.astype(jnp.int32) % BAR_VOC).reshape(BL) + _BAR_OFF
    pi = pos.reshape(BL).astype(jnp.int32) + _POS_OFF
    sidx = jnp.stack([vi, di, bi, pi], axis=0)

    out = _sc_gather_sum(word_proj, small_proj, tok, sidx)
    return out.reshape(B, L, d)
